# baseline (speedup 1.0000x reference)
import functools

import jax
import jax.numpy as jnp
import numpy as np
from jax import lax
from jax.experimental import pallas as pl
from jax.experimental.pallas import tpu as pltpu
from jax.experimental.pallas import tpu_sc as plsc

N_G = 50
CUT = 5.0
LOG2 = 0.6931471805599453


_BE = 3200


def _edge_filter_body(e_ref, w1_ref, b1_ref, w2_ref, b2_ref, out_ref):
    width = CUT / (N_G - 1)
    coeff = -0.5 / (width * width)
    offs = lax.broadcasted_iota(jnp.int32, (1, N_G), 1).astype(jnp.float32) * width
    e = e_ref[...]
    d = e - offs
    eg = jnp.exp(coeff * d * d)
    h = jnp.dot(eg, w1_ref[...], preferred_element_type=jnp.float32) + b1_ref[...]
    h = jax.nn.softplus(h) - LOG2
    out_ref[...] = (
        jnp.dot(h, w2_ref[...], preferred_element_type=jnp.float32) + b2_ref[...]
    )


def _edge_filters(e, W_df1, b_df1, W_df2, b_df2):
    E = e.shape[0]
    grid = E // _BE
    return pl.pallas_call(
        _edge_filter_body,
        grid=(grid,),
        in_specs=[
            pl.BlockSpec((_BE, 1), lambda i: (i, 0)),
            pl.BlockSpec((N_G, N_G), lambda i: (0, 0)),
            pl.BlockSpec((1, N_G), lambda i: (0, 0)),
            pl.BlockSpec((N_G, 128), lambda i: (0, 0)),
            pl.BlockSpec((1, 128), lambda i: (0, 0)),
        ],
        out_specs=pl.BlockSpec((_BE, 128), lambda i: (i, 0)),
        out_shape=jax.ShapeDtypeStruct((E, 128), jnp.float32),
    )(e, W_df1, b_df1.reshape(1, N_G), W_df2, b_df2.reshape(1, 128))


_BN = 2000


def _atom_filter_body(r_ref, w_ref, out_ref):
    out_ref[...] = jnp.dot(r_ref[...], w_ref[...], preferred_element_type=jnp.float32)


def _atom_filter(r, W_af):
    N = r.shape[0]
    grid = N // _BN
    return pl.pallas_call(
        _atom_filter_body,
        grid=(grid,),
        in_specs=[
            pl.BlockSpec((_BN, 128), lambda i: (i, 0)),
            pl.BlockSpec((128, 128), lambda i: (0, 0)),
        ],
        out_specs=pl.BlockSpec((_BN, 128), lambda i: (i, 0)),
        out_shape=jax.ShapeDtypeStruct((N, 128), jnp.float32),
    )(r, W_af)


_C = 40
_NPAD = 10240
_RPT = _NPAD // 16
_EPT = 10000
_CPT = _EPT // _C
_GEN = 3


def _sc_body(a0_hbm, a1_hbm, rf_hbm, eg_hbm, out_hbm, *refs):
    cid = lax.axis_index("c")
    sid = lax.axis_index("s")
    wid = sid * 2 + cid
    ebase = wid * _EPT

    bufs = [refs[7 * g:7 * g + 7] for g in range(_GEN)]
    acc_sh = refs[7 * _GEN]

    def fire(c, b, drain):
        idx0, idx1, rows0, rows1, egb, gsem, ssem = b
        if drain:
            pltpu.make_async_copy(rows0, acc_sh.at[idx1], ssem).wait()
            pltpu.make_async_copy(rows1, acc_sh.at[idx0], ssem).wait()
        base = ebase + c * _C
        pltpu.sync_copy(a0_hbm.at[pl.ds(base, _C)], idx0)
        pltpu.sync_copy(a1_hbm.at[pl.ds(base, _C)], idx1)
        pltpu.async_copy(rf_hbm.at[idx0], rows0, gsem)
        pltpu.async_copy(rf_hbm.at[idx1], rows1, gsem)
        pltpu.async_copy(eg_hbm.at[pl.ds(base, _C)], egb, gsem)

    def process(b):
        idx0, idx1, rows0, rows1, egb, gsem, ssem = b
        pltpu.make_async_copy(rf_hbm.at[idx0], rows0, gsem).wait()
        pltpu.make_async_copy(rf_hbm.at[idx1], rows1, gsem).wait()
        pltpu.make_async_copy(eg_hbm.at[pl.ds(0, _C)], egb, gsem).wait()

        @pl.loop(0, _C, unroll=2)
        def _(i):
            for j in range(8):
                s = pl.ds(j * 16, 16)
                eij = egb[i, s]
                rows0[i, s] = rows0[i, s] * eij
                rows1[i, s] = rows1[i, s] * eij

        pltpu.async_copy(rows0, acc_sh.at[idx1], ssem, add=True)
        pltpu.async_copy(rows1, acc_sh.at[idx0], ssem, add=True)

    def drain_scatters(b):
        idx0, idx1, rows0, rows1, egb, gsem, ssem = b
        pltpu.make_async_copy(rows0, acc_sh.at[idx1], ssem).wait()
        pltpu.make_async_copy(rows1, acc_sh.at[idx0], ssem).wait()

    zbuf = bufs[0][2]
    zeros16 = jnp.zeros((16,), jnp.float32)

    @pl.loop(0, _C)
    def _(i):
        for j in range(8):
            zbuf[i, pl.ds(j * 16, 16)] = zeros16

    @pl.loop(0, _RPT // _C)
    def _(k):
        pltpu.sync_copy(zbuf, acc_sh.at[pl.ds(sid * _RPT + k * _C, _C)])

    plsc.subcore_barrier()

    fire(0, bufs[0], False)
    fire(1, bufs[1], False)
    fire(2, bufs[2], False)

    @pl.loop(0, (_CPT - _GEN - 1) // _GEN)
    def _(kk):
        for r in range(_GEN):
            c = _GEN * kk + r
            process(bufs[r])
            fire(c + _GEN, bufs[r], True)

    process(bufs[0])
    fire(_CPT - 1, bufs[0], True)
    process(bufs[1])
    process(bufs[2])
    process(bufs[0])
    drain_scatters(bufs[1])
    drain_scatters(bufs[2])
    drain_scatters(bufs[0])

    plsc.subcore_barrier()

    @pl.loop(0, _RPT // _C)
    def _(k):
        r0 = sid * _RPT + k * _C
        pltpu.sync_copy(acc_sh.at[pl.ds(r0, _C)], zbuf)
        pltpu.sync_copy(zbuf, out_hbm.at[pl.ds(cid * _NPAD + r0, _C)])


def _sc_aggregate(a0, a1, rf, eg):
    mesh = plsc.VectorSubcoreMesh(core_axis_name="c", subcore_axis_name="s")
    gen_scratch = [
        pltpu.VMEM((_C,), jnp.int32),
        pltpu.VMEM((_C,), jnp.int32),
        pltpu.VMEM((_C, 128), jnp.float32),
        pltpu.VMEM((_C, 128), jnp.float32),
        pltpu.VMEM((_C, 128), jnp.float32),
        pltpu.SemaphoreType.DMA,
        pltpu.SemaphoreType.DMA,
    ]
    k = pl.kernel(
        _sc_body,
        out_type=jax.ShapeDtypeStruct((2 * _NPAD, 128), jnp.float32),
        mesh=mesh,
        scratch_types=gen_scratch * _GEN
        + [pltpu.VMEM_SHARED((_NPAD, 128), jnp.float32)],
    )
    return k(a0, a1, rf, eg)


_BU = 400


def _update_body(p_ref, w1_ref, b1_ref, w2_ref, b2_ref, out_ref):
    agg = p_ref[0] + p_ref[1]
    h = jnp.dot(agg, w1_ref[...], preferred_element_type=jnp.float32) + b1_ref[...]
    h = jax.nn.softplus(h) - LOG2
    out_ref[...] = (
        jnp.dot(h, w2_ref[...], preferred_element_type=jnp.float32) + b2_ref[...]
    )


def _node_update(parts, W_d1, b_d1, W_d2, b_d2, N):
    grid = N // _BU
    return pl.pallas_call(
        _update_body,
        grid=(grid,),
        in_specs=[
            pl.BlockSpec((2, _BU, 128), lambda i: (0, i, 0)),
            pl.BlockSpec((128, 128), lambda i: (0, 0)),
            pl.BlockSpec((1, 128), lambda i: (0, 0)),
            pl.BlockSpec((128, 128), lambda i: (0, 0)),
            pl.BlockSpec((1, 128), lambda i: (0, 0)),
        ],
        out_specs=pl.BlockSpec((_BU, 128), lambda i: (i, 0)),
        out_shape=jax.ShapeDtypeStruct((N, 128), jnp.float32),
    )(parts, W_d1, b_d1.reshape(1, 128), W_d2, b_d2.reshape(1, 128))


@jax.jit
def kernel(r, e, a, W_df1, b_df1, W_df2, b_df2, W_af, W_d1, b_d1, W_d2, b_d2):
    N = r.shape[0]
    eg = _edge_filters(e, W_df1, b_df1, W_df2, b_df2)
    rf = _atom_filter(r, W_af)
    a0 = a[:, 0]
    a1 = a[:, 1]
    parts_flat = _sc_aggregate(a0, a1, rf, eg)
    parts = parts_flat.reshape(2, _NPAD, 128)
    return _node_update(parts, W_d1, b_d1, W_d2, b_d2, N)


# speedup vs baseline: 1.1491x; 1.1491x over previous
import functools

import jax
import jax.numpy as jnp
import numpy as np
from jax import lax
from jax.experimental import pallas as pl
from jax.experimental.pallas import tpu as pltpu
from jax.experimental.pallas import tpu_sc as plsc

N_G = 50
CUT = 5.0
LOG2 = 0.6931471805599453


_BE = 3200


def _edge_filter_body(e_ref, w1_ref, b1_ref, w2_ref, b2_ref, out_ref):
    width = CUT / (N_G - 1)
    coeff = -0.5 / (width * width)
    offs = lax.broadcasted_iota(jnp.int32, (1, N_G), 1).astype(jnp.float32) * width
    e = e_ref[...]
    d = e - offs
    eg = jnp.exp(coeff * d * d)
    h = jnp.dot(eg, w1_ref[...], preferred_element_type=jnp.float32) + b1_ref[...]
    h = jax.nn.softplus(h) - LOG2
    out_ref[...] = (
        jnp.dot(h, w2_ref[...], preferred_element_type=jnp.float32) + b2_ref[...]
    )


def _edge_filters(e, W_df1, b_df1, W_df2, b_df2):
    E = e.shape[0]
    grid = E // _BE
    return pl.pallas_call(
        _edge_filter_body,
        grid=(grid,),
        in_specs=[
            pl.BlockSpec((_BE, 1), lambda i: (i, 0)),
            pl.BlockSpec((N_G, N_G), lambda i: (0, 0)),
            pl.BlockSpec((1, N_G), lambda i: (0, 0)),
            pl.BlockSpec((N_G, 128), lambda i: (0, 0)),
            pl.BlockSpec((1, 128), lambda i: (0, 0)),
        ],
        out_specs=pl.BlockSpec((_BE, 128), lambda i: (i, 0)),
        out_shape=jax.ShapeDtypeStruct((E, 128), jnp.float32),
    )(e, W_df1, b_df1.reshape(1, N_G), W_df2, b_df2.reshape(1, 128))


_BN = 2000


def _atom_filter_body(r_ref, w_ref, out_ref):
    out_ref[...] = jnp.dot(r_ref[...], w_ref[...], preferred_element_type=jnp.float32)


def _atom_filter(r, W_af):
    N = r.shape[0]
    grid = N // _BN
    return pl.pallas_call(
        _atom_filter_body,
        grid=(grid,),
        in_specs=[
            pl.BlockSpec((_BN, 128), lambda i: (i, 0)),
            pl.BlockSpec((128, 128), lambda i: (0, 0)),
        ],
        out_specs=pl.BlockSpec((_BN, 128), lambda i: (i, 0)),
        out_shape=jax.ShapeDtypeStruct((N, 128), jnp.float32),
    )(r, W_af)


_C = 40
_NACC = 10000
_EPT = 10000
_CPT = _EPT // _C
_NBLK = _NACC // _C


def _sc_body(ap_hbm, rf_hbm, eg_hbm, out_hbm,
             ap_f,
             idx0_a, idx1_a, rows0_a, rows1_a, eg_a,
             idx0_b, idx1_b, rows0_b, rows1_b, eg_b,
             acc_sh, sem_a, sem_b):
    cid = lax.axis_index("c")
    sid = lax.axis_index("s")
    wid = sid * 2 + cid

    bufs_a = (idx0_a, idx1_a, rows0_a, rows1_a, eg_a, sem_a)
    bufs_b = (idx0_b, idx1_b, rows0_b, rows1_b, eg_b, sem_b)

    def fire(c, bufs):
        idx0, idx1, rows0, rows1, egb, sem = bufs
        base = c * _C
        for off in (0, 16, 24):
            s = pl.ds(off, 16)
            w = ap_f[pl.ds(base + off, 16)]
            idx0[s] = (w & 0xFFFF).astype(jnp.int32)
            idx1[s] = (w >> 16).astype(jnp.int32)
        pltpu.async_copy(rf_hbm.at[idx0], rows0, sem)
        pltpu.async_copy(rf_hbm.at[idx1], rows1, sem)
        pltpu.async_copy(eg_hbm.at[pl.ds(wid * _EPT + base, _C)], egb, sem)

    def process(c, bufs):
        idx0, idx1, rows0, rows1, egb, sem = bufs
        pltpu.make_async_copy(rf_hbm.at[idx0], rows0, sem).wait()
        pltpu.make_async_copy(rf_hbm.at[idx0], rows1, sem).wait()
        pltpu.make_async_copy(eg_hbm.at[pl.ds(0, _C)], egb, sem).wait()

        @pl.loop(0, _C, unroll=2)
        def _(i):
            for j in range(8):
                s = pl.ds(j * 16, 16)
                eij = egb[i, s]
                rows0[i, s] = rows0[i, s] * eij
                rows1[i, s] = rows1[i, s] * eij

        pltpu.sync_copy(rows0, acc_sh.at[idx1], add=True)
        pltpu.sync_copy(rows1, acc_sh.at[idx0], add=True)

    pltpu.sync_copy(ap_hbm.at[pl.ds(wid * _EPT, _EPT)], ap_f)

    zeros16 = jnp.zeros((16,), jnp.float32)

    @pl.loop(0, _C)
    def _(i):
        for j in range(8):
            rows0_a[i, pl.ds(j * 16, 16)] = zeros16

    nz = (_NBLK - sid + 15) // 16

    @pl.loop(0, nz)
    def _(k):
        pltpu.sync_copy(rows0_a, acc_sh.at[pl.ds((sid + k * 16) * _C, _C)])

    plsc.subcore_barrier()

    fire(0, bufs_a)

    @pl.loop(0, _CPT // 2 - 1)
    def _(kk):
        fire(2 * kk + 1, bufs_b)
        process(2 * kk, bufs_a)
        fire(2 * kk + 2, bufs_a)
        process(2 * kk + 1, bufs_b)

    fire(_CPT - 1, bufs_b)
    process(_CPT - 2, bufs_a)
    process(_CPT - 1, bufs_b)

    plsc.subcore_barrier()

    @pl.loop(0, nz)
    def _(k):
        r0 = (sid + k * 16) * _C
        pltpu.sync_copy(acc_sh.at[pl.ds(r0, _C)], rows0_a)
        pltpu.sync_copy(rows0_a, out_hbm.at[pl.ds(cid * _NACC + r0, _C)])


def _sc_aggregate(ap, rf, eg):
    mesh = plsc.VectorSubcoreMesh(core_axis_name="c", subcore_axis_name="s")
    k = pl.kernel(
        _sc_body,
        out_type=jax.ShapeDtypeStruct((2 * _NACC, 128), jnp.float32),
        mesh=mesh,
        scratch_types=[
            pltpu.VMEM((_EPT,), jnp.uint32),
            pltpu.VMEM((_C,), jnp.int32),
            pltpu.VMEM((_C,), jnp.int32),
            pltpu.VMEM((_C, 128), jnp.float32),
            pltpu.VMEM((_C, 128), jnp.float32),
            pltpu.VMEM((_C, 128), jnp.float32),
            pltpu.VMEM((_C,), jnp.int32),
            pltpu.VMEM((_C,), jnp.int32),
            pltpu.VMEM((_C, 128), jnp.float32),
            pltpu.VMEM((_C, 128), jnp.float32),
            pltpu.VMEM((_C, 128), jnp.float32),
            pltpu.VMEM_SHARED((_NACC, 128), jnp.float32),
            pltpu.SemaphoreType.DMA,
            pltpu.SemaphoreType.DMA,
        ],
    )
    return k(ap, rf, eg)


_BU = 400


def _update_body(p_ref, w1_ref, b1_ref, w2_ref, b2_ref, out_ref):
    agg = p_ref[0] + p_ref[1]
    h = jnp.dot(agg, w1_ref[...], preferred_element_type=jnp.float32) + b1_ref[...]
    h = jax.nn.softplus(h) - LOG2
    out_ref[...] = (
        jnp.dot(h, w2_ref[...], preferred_element_type=jnp.float32) + b2_ref[...]
    )


def _node_update(parts, W_d1, b_d1, W_d2, b_d2, N):
    grid = N // _BU
    return pl.pallas_call(
        _update_body,
        grid=(grid,),
        in_specs=[
            pl.BlockSpec((2, _BU, 128), lambda i: (0, i, 0)),
            pl.BlockSpec((128, 128), lambda i: (0, 0)),
            pl.BlockSpec((1, 128), lambda i: (0, 0)),
            pl.BlockSpec((128, 128), lambda i: (0, 0)),
            pl.BlockSpec((1, 128), lambda i: (0, 0)),
        ],
        out_specs=pl.BlockSpec((_BU, 128), lambda i: (i, 0)),
        out_shape=jax.ShapeDtypeStruct((N, 128), jnp.float32),
    )(parts, W_d1, b_d1.reshape(1, 128), W_d2, b_d2.reshape(1, 128))


@jax.jit
def kernel(r, e, a, W_df1, b_df1, W_df2, b_df2, W_af, W_d1, b_d1, W_d2, b_d2):
    N = r.shape[0]
    eg = _edge_filters(e, W_df1, b_df1, W_df2, b_df2)
    rf = _atom_filter(r, W_af)
    ap = a[:, 0].astype(jnp.uint32) | (a[:, 1].astype(jnp.uint32) << 16)
    parts_flat = _sc_aggregate(ap, rf, eg)
    parts = parts_flat.reshape(2, _NACC, 128)
    return _node_update(parts, W_d1, b_d1, W_d2, b_d2, N)


# speedup vs baseline: 1.3436x; 1.1692x over previous
import functools

import jax
import jax.numpy as jnp
import numpy as np
from jax import lax
from jax.experimental import pallas as pl
from jax.experimental.pallas import tpu as pltpu
from jax.experimental.pallas import tpu_sc as plsc

N_G = 50
CUT = 5.0
LOG2 = 0.6931471805599453


_BE = 3200


def _edge_filter_body(e_ref, w1_ref, b1_ref, w2_ref, b2_ref, out_ref):
    width = CUT / (N_G - 1)
    coeff = -0.5 / (width * width)
    offs = lax.broadcasted_iota(jnp.int32, (1, N_G), 1).astype(jnp.float32) * width
    e = e_ref[...]
    d = e - offs
    eg = jnp.exp(coeff * d * d)
    h = jnp.dot(eg, w1_ref[...], preferred_element_type=jnp.float32) + b1_ref[...]
    h = jax.nn.softplus(h) - LOG2
    out_ref[...] = (
        jnp.dot(h, w2_ref[...], preferred_element_type=jnp.float32) + b2_ref[...]
    )


def _edge_filters(e, W_df1, b_df1, W_df2, b_df2):
    E = e.shape[0]
    grid = E // _BE
    return pl.pallas_call(
        _edge_filter_body,
        grid=(grid,),
        in_specs=[
            pl.BlockSpec((_BE, 1), lambda i: (i, 0)),
            pl.BlockSpec((N_G, N_G), lambda i: (0, 0)),
            pl.BlockSpec((1, N_G), lambda i: (0, 0)),
            pl.BlockSpec((N_G, 128), lambda i: (0, 0)),
            pl.BlockSpec((1, 128), lambda i: (0, 0)),
        ],
        out_specs=pl.BlockSpec((_BE, 128), lambda i: (i, 0)),
        out_shape=jax.ShapeDtypeStruct((E, 128), jnp.float32),
    )(e, W_df1, b_df1.reshape(1, N_G), W_df2, b_df2.reshape(1, 128))


_BN = 2000


def _atom_filter_body(r_ref, w_ref, out_ref):
    out_ref[...] = jnp.dot(r_ref[...], w_ref[...], preferred_element_type=jnp.float32)


def _atom_filter(r, W_af):
    N = r.shape[0]
    grid = N // _BN
    return pl.pallas_call(
        _atom_filter_body,
        grid=(grid,),
        in_specs=[
            pl.BlockSpec((_BN, 128), lambda i: (i, 0)),
            pl.BlockSpec((128, 128), lambda i: (0, 0)),
        ],
        out_specs=pl.BlockSpec((_BN, 128), lambda i: (i, 0)),
        out_shape=jax.ShapeDtypeStruct((N, 128), jnp.float32),
    )(r, W_af)


_C = 40
_NACC = 10000
_EPT = 10000
_CPT = _EPT // _C
_NBLK = _NACC // _C


def _sc_body(a0_hbm, a1_hbm, rf_hbm, eg_hbm, out_hbm,
             idx0_a, idx1_a, sidx_a, rows_a, eg_a,
             idx0_b, idx1_b, sidx_b, rows_b, eg_b,
             acc_sh, sem_a, sem_b):
    cid = lax.axis_index("c")
    sid = lax.axis_index("s")
    wid = sid * 2 + cid
    ebase = wid * _EPT

    bufs_a = (idx0_a, idx1_a, sidx_a, rows_a, eg_a, sem_a)
    bufs_b = (idx0_b, idx1_b, sidx_b, rows_b, eg_b, sem_b)

    def fire(c, bufs):
        idx0, idx1, sidx, rows, egb, sem = bufs
        base = ebase + c * _C
        pltpu.sync_copy(a0_hbm.at[pl.ds(base, _C)], idx0)
        pltpu.sync_copy(a1_hbm.at[pl.ds(base, _C)], idx1)
        pltpu.async_copy(rf_hbm.at[idx0], rows.at[pl.ds(0, _C)], sem)
        pltpu.async_copy(rf_hbm.at[idx1], rows.at[pl.ds(_C, _C)], sem)
        pltpu.async_copy(eg_hbm.at[pl.ds(base, _C)], egb, sem)

    def process(c, bufs):
        idx0, idx1, sidx, rows, egb, sem = bufs
        pltpu.make_async_copy(rf_hbm.at[idx0], rows.at[pl.ds(0, _C)], sem).wait()
        pltpu.make_async_copy(rf_hbm.at[idx0], rows.at[pl.ds(_C, _C)], sem).wait()
        pltpu.make_async_copy(eg_hbm.at[pl.ds(0, _C)], egb, sem).wait()

        for off in (0, 16, 24):
            sidx[pl.ds(off, 16)] = idx1[pl.ds(off, 16)]
            sidx[pl.ds(_C + off, 16)] = idx0[pl.ds(off, 16)]

        @pl.loop(0, _C, unroll=2)
        def _(i):
            for j in range(8):
                s = pl.ds(j * 16, 16)
                eij = egb[i, s]
                rows[i, s] = rows[i, s] * eij
                rows[_C + i, s] = rows[_C + i, s] * eij

        pltpu.sync_copy(rows, acc_sh.at[sidx], add=True)

    zeros16 = jnp.zeros((16,), jnp.float32)

    @pl.loop(0, 2 * _C)
    def _(i):
        for j in range(8):
            rows_a[i, pl.ds(j * 16, 16)] = zeros16

    nz = (_NACC // (2 * _C) - sid + 15) // 16

    @pl.loop(0, nz)
    def _(k):
        pltpu.sync_copy(rows_a, acc_sh.at[pl.ds((sid + k * 16) * 2 * _C, 2 * _C)])

    plsc.subcore_barrier()

    fire(0, bufs_a)

    @pl.loop(0, _CPT // 2 - 1)
    def _(kk):
        fire(2 * kk + 1, bufs_b)
        process(2 * kk, bufs_a)
        fire(2 * kk + 2, bufs_a)
        process(2 * kk + 1, bufs_b)

    fire(_CPT - 1, bufs_b)
    process(_CPT - 2, bufs_a)
    process(_CPT - 1, bufs_b)

    plsc.subcore_barrier()

    @pl.loop(0, nz)
    def _(k):
        r0 = (sid + k * 16) * 2 * _C
        pltpu.sync_copy(acc_sh.at[pl.ds(r0, 2 * _C)], rows_a)
        pltpu.sync_copy(rows_a, out_hbm.at[pl.ds(cid * _NACC + r0, 2 * _C)])


def _sc_aggregate(a0, a1, rf, eg):
    mesh = plsc.VectorSubcoreMesh(core_axis_name="c", subcore_axis_name="s")
    k = pl.kernel(
        _sc_body,
        out_type=jax.ShapeDtypeStruct((2 * _NACC, 128), jnp.float32),
        mesh=mesh,
        scratch_types=[
            pltpu.VMEM((_C,), jnp.int32),
            pltpu.VMEM((_C,), jnp.int32),
            pltpu.VMEM((2 * _C,), jnp.int32),
            pltpu.VMEM((2 * _C, 128), jnp.float32),
            pltpu.VMEM((_C, 128), jnp.float32),
            pltpu.VMEM((_C,), jnp.int32),
            pltpu.VMEM((_C,), jnp.int32),
            pltpu.VMEM((2 * _C,), jnp.int32),
            pltpu.VMEM((2 * _C, 128), jnp.float32),
            pltpu.VMEM((_C, 128), jnp.float32),
            pltpu.VMEM_SHARED((_NACC, 128), jnp.float32),
            pltpu.SemaphoreType.DMA,
            pltpu.SemaphoreType.DMA,
        ],
    )
    return k(a0, a1, rf, eg)


_BU = 400


def _update_body(p_ref, w1_ref, b1_ref, w2_ref, b2_ref, out_ref):
    agg = p_ref[0] + p_ref[1]
    h = jnp.dot(agg, w1_ref[...], preferred_element_type=jnp.float32) + b1_ref[...]
    h = jax.nn.softplus(h) - LOG2
    out_ref[...] = (
        jnp.dot(h, w2_ref[...], preferred_element_type=jnp.float32) + b2_ref[...]
    )


def _node_update(parts, W_d1, b_d1, W_d2, b_d2, N):
    grid = N // _BU
    return pl.pallas_call(
        _update_body,
        grid=(grid,),
        in_specs=[
            pl.BlockSpec((2, _BU, 128), lambda i: (0, i, 0)),
            pl.BlockSpec((128, 128), lambda i: (0, 0)),
            pl.BlockSpec((1, 128), lambda i: (0, 0)),
            pl.BlockSpec((128, 128), lambda i: (0, 0)),
            pl.BlockSpec((1, 128), lambda i: (0, 0)),
        ],
        out_specs=pl.BlockSpec((_BU, 128), lambda i: (i, 0)),
        out_shape=jax.ShapeDtypeStruct((N, 128), jnp.float32),
    )(parts, W_d1, b_d1.reshape(1, 128), W_d2, b_d2.reshape(1, 128))


@jax.jit
def kernel(r, e, a, W_df1, b_df1, W_df2, b_df2, W_af, W_d1, b_d1, W_d2, b_d2):
    N = r.shape[0]
    eg = _edge_filters(e, W_df1, b_df1, W_df2, b_df2)
    rf = _atom_filter(r, W_af)
    parts_flat = _sc_aggregate(a[:, 0], a[:, 1], rf, eg)
    parts = parts_flat.reshape(2, _NACC, 128)
    return _node_update(parts, W_d1, b_d1, W_d2, b_d2, N)


# speedup vs baseline: 1.3451x; 1.0011x over previous
import functools

import jax
import jax.numpy as jnp
import numpy as np
from jax import lax
from jax.experimental import pallas as pl
from jax.experimental.pallas import tpu as pltpu
from jax.experimental.pallas import tpu_sc as plsc

N_G = 50
CUT = 5.0
LOG2 = 0.6931471805599453


_BE = 3200


def _edge_filter_body(e_ref, w1_ref, b1_ref, w2_ref, b2_ref, out_ref):
    width = CUT / (N_G - 1)
    coeff = -0.5 / (width * width)
    offs = lax.broadcasted_iota(jnp.int32, (1, N_G), 1).astype(jnp.float32) * width
    e = e_ref[...]
    d = e - offs
    eg = jnp.exp(coeff * d * d)
    h = jnp.dot(eg, w1_ref[...], preferred_element_type=jnp.float32) + b1_ref[...]
    h = jax.nn.softplus(h) - LOG2
    out_ref[...] = (
        jnp.dot(h, w2_ref[...], preferred_element_type=jnp.float32) + b2_ref[...]
    )


def _edge_filters(e, W_df1, b_df1, W_df2, b_df2):
    E = e.shape[0]
    grid = E // _BE
    return pl.pallas_call(
        _edge_filter_body,
        grid=(grid,),
        in_specs=[
            pl.BlockSpec((_BE, 1), lambda i: (i, 0)),
            pl.BlockSpec((N_G, N_G), lambda i: (0, 0)),
            pl.BlockSpec((1, N_G), lambda i: (0, 0)),
            pl.BlockSpec((N_G, 128), lambda i: (0, 0)),
            pl.BlockSpec((1, 128), lambda i: (0, 0)),
        ],
        out_specs=pl.BlockSpec((_BE, 128), lambda i: (i, 0)),
        out_shape=jax.ShapeDtypeStruct((E, 128), jnp.float32),
    )(e, W_df1, b_df1.reshape(1, N_G), W_df2, b_df2.reshape(1, 128))


_BN = 2000


def _atom_filter_body(r_ref, w_ref, out_ref):
    out_ref[...] = jnp.dot(r_ref[...], w_ref[...], preferred_element_type=jnp.float32)


def _atom_filter(r, W_af):
    N = r.shape[0]
    grid = N // _BN
    return pl.pallas_call(
        _atom_filter_body,
        grid=(grid,),
        in_specs=[
            pl.BlockSpec((_BN, 128), lambda i: (i, 0)),
            pl.BlockSpec((128, 128), lambda i: (0, 0)),
        ],
        out_specs=pl.BlockSpec((_BN, 128), lambda i: (i, 0)),
        out_shape=jax.ShapeDtypeStruct((N, 128), jnp.float32),
    )(r, W_af)


_C = 40
_NACC = 10000
_EPT = 10000
_CPT = _EPT // _C
_NBLK = _NACC // _C


def _sc_body(a0_hbm, a1_hbm, rf_hbm, eg_hbm, out_hbm,
             sidx_a, rows_a, eg_a,
             sidx_b, rows_b, eg_b,
             acc_sh, sem_a, sem_b):
    cid = lax.axis_index("c")
    sid = lax.axis_index("s")
    wid = sid * 2 + cid
    ebase = wid * _EPT

    bufs_a = (sidx_a, rows_a, eg_a, sem_a)
    bufs_b = (sidx_b, rows_b, eg_b, sem_b)

    def fire(c, bufs):
        sidx, rows, egb, sem = bufs
        base = ebase + c * _C
        pltpu.sync_copy(a1_hbm.at[pl.ds(base, _C)], sidx.at[pl.ds(0, _C)])
        pltpu.sync_copy(a0_hbm.at[pl.ds(base, _C)], sidx.at[pl.ds(_C, _C)])
        pltpu.async_copy(rf_hbm.at[sidx.at[pl.ds(_C, _C)]], rows.at[pl.ds(0, _C)], sem)
        pltpu.async_copy(rf_hbm.at[sidx.at[pl.ds(0, _C)]], rows.at[pl.ds(_C, _C)], sem)
        pltpu.async_copy(eg_hbm.at[pl.ds(base, _C)], egb, sem)

    def process(c, bufs):
        sidx, rows, egb, sem = bufs
        pltpu.make_async_copy(eg_hbm.at[pl.ds(0, _C)], rows.at[pl.ds(0, _C)], sem).wait()
        pltpu.make_async_copy(eg_hbm.at[pl.ds(0, _C)], rows.at[pl.ds(_C, _C)], sem).wait()
        pltpu.make_async_copy(eg_hbm.at[pl.ds(0, _C)], egb, sem).wait()

        @pl.loop(0, _C, unroll=2)
        def _(i):
            for j in range(8):
                s = pl.ds(j * 16, 16)
                eij = egb[i, s]
                rows[i, s] = rows[i, s] * eij
                rows[_C + i, s] = rows[_C + i, s] * eij

        pltpu.sync_copy(rows, acc_sh.at[sidx], add=True)

    zeros16 = jnp.zeros((16,), jnp.float32)

    @pl.loop(0, 2 * _C)
    def _(i):
        for j in range(8):
            rows_a[i, pl.ds(j * 16, 16)] = zeros16

    nz = (_NACC // (2 * _C) - sid + 15) // 16

    @pl.loop(0, nz)
    def _(k):
        pltpu.sync_copy(rows_a, acc_sh.at[pl.ds((sid + k * 16) * 2 * _C, 2 * _C)])

    plsc.subcore_barrier()

    fire(0, bufs_a)

    @pl.loop(0, _CPT // 2 - 1)
    def _(kk):
        fire(2 * kk + 1, bufs_b)
        process(2 * kk, bufs_a)
        fire(2 * kk + 2, bufs_a)
        process(2 * kk + 1, bufs_b)

    fire(_CPT - 1, bufs_b)
    process(_CPT - 2, bufs_a)
    process(_CPT - 1, bufs_b)

    plsc.subcore_barrier()

    @pl.loop(0, nz)
    def _(k):
        r0 = (sid + k * 16) * 2 * _C
        pltpu.sync_copy(acc_sh.at[pl.ds(r0, 2 * _C)], rows_a)
        pltpu.sync_copy(rows_a, out_hbm.at[pl.ds(cid * _NACC + r0, 2 * _C)])


def _sc_aggregate(a0, a1, rf, eg):
    mesh = plsc.VectorSubcoreMesh(core_axis_name="c", subcore_axis_name="s")
    k = pl.kernel(
        _sc_body,
        out_type=jax.ShapeDtypeStruct((2 * _NACC, 128), jnp.float32),
        mesh=mesh,
        scratch_types=[
            pltpu.VMEM((2 * _C,), jnp.int32),
            pltpu.VMEM((2 * _C, 128), jnp.float32),
            pltpu.VMEM((_C, 128), jnp.float32),
            pltpu.VMEM((2 * _C,), jnp.int32),
            pltpu.VMEM((2 * _C, 128), jnp.float32),
            pltpu.VMEM((_C, 128), jnp.float32),
            pltpu.VMEM_SHARED((_NACC, 128), jnp.float32),
            pltpu.SemaphoreType.DMA,
            pltpu.SemaphoreType.DMA,
        ],
    )
    return k(a0, a1, rf, eg)


_BU = 400


def _update_body(p_ref, w1_ref, b1_ref, w2_ref, b2_ref, out_ref):
    agg = p_ref[0] + p_ref[1]
    h = jnp.dot(agg, w1_ref[...], preferred_element_type=jnp.float32) + b1_ref[...]
    h = jax.nn.softplus(h) - LOG2
    out_ref[...] = (
        jnp.dot(h, w2_ref[...], preferred_element_type=jnp.float32) + b2_ref[...]
    )


def _node_update(parts, W_d1, b_d1, W_d2, b_d2, N):
    grid = N // _BU
    return pl.pallas_call(
        _update_body,
        grid=(grid,),
        in_specs=[
            pl.BlockSpec((2, _BU, 128), lambda i: (0, i, 0)),
            pl.BlockSpec((128, 128), lambda i: (0, 0)),
            pl.BlockSpec((1, 128), lambda i: (0, 0)),
            pl.BlockSpec((128, 128), lambda i: (0, 0)),
            pl.BlockSpec((1, 128), lambda i: (0, 0)),
        ],
        out_specs=pl.BlockSpec((_BU, 128), lambda i: (i, 0)),
        out_shape=jax.ShapeDtypeStruct((N, 128), jnp.float32),
    )(parts, W_d1, b_d1.reshape(1, 128), W_d2, b_d2.reshape(1, 128))


@jax.jit
def kernel(r, e, a, W_df1, b_df1, W_df2, b_df2, W_af, W_d1, b_d1, W_d2, b_d2):
    N = r.shape[0]
    eg = _edge_filters(e, W_df1, b_df1, W_df2, b_df2)
    rf = _atom_filter(r, W_af)
    parts_flat = _sc_aggregate(a[:, 0], a[:, 1], rf, eg)
    parts = parts_flat.reshape(2, _NACC, 128)
    return _node_update(parts, W_d1, b_d1, W_d2, b_d2, N)


# speedup vs baseline: 1.8044x; 1.3414x over previous
import functools

import jax
import jax.numpy as jnp
import numpy as np
from jax import lax
from jax.experimental import pallas as pl
from jax.experimental.pallas import tpu as pltpu
from jax.experimental.pallas import tpu_sc as plsc

N_G = 50
CUT = 5.0
LOG2 = 0.6931471805599453


_BE = 3200


def _edge_filter_body(e_ref, w1_ref, b1_ref, w2_ref, b2_ref, out_ref):
    width = CUT / (N_G - 1)
    coeff = -0.5 / (width * width)
    offs = lax.broadcasted_iota(jnp.int32, (1, N_G), 1).astype(jnp.float32) * width
    e = e_ref[...]
    d = e - offs
    eg = jnp.exp(coeff * d * d)
    h = jnp.dot(eg, w1_ref[...], preferred_element_type=jnp.float32) + b1_ref[...]
    h = jax.nn.softplus(h) - LOG2
    out_ref[...] = (
        jnp.dot(h, w2_ref[...], preferred_element_type=jnp.float32) + b2_ref[...]
    )


def _edge_filters(e, W_df1, b_df1, W_df2, b_df2):
    E = e.shape[0]
    grid = E // _BE
    return pl.pallas_call(
        _edge_filter_body,
        grid=(grid,),
        in_specs=[
            pl.BlockSpec((_BE, 1), lambda i: (i, 0)),
            pl.BlockSpec((N_G, N_G), lambda i: (0, 0)),
            pl.BlockSpec((1, N_G), lambda i: (0, 0)),
            pl.BlockSpec((N_G, 128), lambda i: (0, 0)),
            pl.BlockSpec((1, 128), lambda i: (0, 0)),
        ],
        out_specs=pl.BlockSpec((_BE, 128), lambda i: (i, 0)),
        out_shape=jax.ShapeDtypeStruct((E, 128), jnp.float32),
    )(e, W_df1, b_df1.reshape(1, N_G), W_df2, b_df2.reshape(1, 128))


_BN = 2000


def _atom_filter_body(r_ref, w_ref, out_ref):
    out_ref[...] = jnp.dot(r_ref[...], w_ref[...], preferred_element_type=jnp.float32)


def _atom_filter(r, W_af):
    N = r.shape[0]
    grid = N // _BN
    return pl.pallas_call(
        _atom_filter_body,
        grid=(grid,),
        in_specs=[
            pl.BlockSpec((_BN, 128), lambda i: (i, 0)),
            pl.BlockSpec((128, 128), lambda i: (0, 0)),
        ],
        out_specs=pl.BlockSpec((_BN, 128), lambda i: (i, 0)),
        out_shape=jax.ShapeDtypeStruct((N, 128), jnp.float32),
    )(r, W_af)


_C = 40
_NACC = 10000
_EPT = 10000
_CPT = _EPT // _C
_NBLK = _NACC // _C


def _sc_body(a0_hbm, a1_hbm, rf_hbm, eg_hbm, out_hbm,
             sidx_a, rows_a, eg_a,
             sidx_b, rows_b, eg_b,
             acc_sh, sem_a, sem_b):
    cid = lax.axis_index("c")
    sid = lax.axis_index("s")
    wid = sid * 2 + cid
    ebase = wid * _EPT

    bufs_a = (sidx_a, rows_a, eg_a, sem_a)
    bufs_b = (sidx_b, rows_b, eg_b, sem_b)

    def fire(c, bufs):
        sidx, rows, egb, sem = bufs
        base = ebase + c * _C
        pltpu.sync_copy(a1_hbm.at[pl.ds(base, _C)], sidx.at[pl.ds(0, _C)])
        pltpu.sync_copy(a0_hbm.at[pl.ds(base, _C)], sidx.at[pl.ds(_C, _C)])
        pltpu.async_copy(rf_hbm.at[sidx.at[pl.ds(_C, _C)]], rows.at[pl.ds(0, _C)], sem)
        pltpu.async_copy(rf_hbm.at[sidx.at[pl.ds(0, _C)]], rows.at[pl.ds(_C, _C)], sem)
        pltpu.async_copy(eg_hbm.at[pl.ds(base, _C)], egb, sem)

    def process(c, bufs):
        sidx, rows, egb, sem = bufs
        pltpu.make_async_copy(eg_hbm.at[pl.ds(0, _C)], rows.at[pl.ds(0, _C)], sem).wait()
        pltpu.make_async_copy(eg_hbm.at[pl.ds(0, _C)], rows.at[pl.ds(_C, _C)], sem).wait()
        pltpu.make_async_copy(eg_hbm.at[pl.ds(0, _C)], egb, sem).wait()

        @pl.loop(0, _C)
        def _(i):
            for j in range(8):
                s = pl.ds(j * 16, 16)
                eij = egb[i, s]
                rows[i, s] = rows[i, s] * eij
                rows[_C + i, s] = rows[_C + i, s] * eij

        pltpu.sync_copy(rows, acc_sh.at[sidx], add=True)

    zeros16 = jnp.zeros((16,), jnp.float32)

    @pl.loop(0, 2 * _C)
    def _(i):
        for j in range(8):
            rows_a[i, pl.ds(j * 16, 16)] = zeros16

    nz = (_NACC // (2 * _C) - sid + 15) // 16

    @pl.loop(0, nz)
    def _(k):
        pltpu.sync_copy(rows_a, acc_sh.at[pl.ds((sid + k * 16) * 2 * _C, 2 * _C)])

    plsc.subcore_barrier()

    fire(0, bufs_a)

    @pl.loop(0, _CPT // 2 - 1)
    def _(kk):
        fire(2 * kk + 1, bufs_b)
        process(2 * kk, bufs_a)
        fire(2 * kk + 2, bufs_a)
        process(2 * kk + 1, bufs_b)

    fire(_CPT - 1, bufs_b)
    process(_CPT - 2, bufs_a)
    process(_CPT - 1, bufs_b)

    plsc.subcore_barrier()

    @pl.loop(0, nz)
    def _(k):
        r0 = (sid + k * 16) * 2 * _C
        pltpu.sync_copy(acc_sh.at[pl.ds(r0, 2 * _C)], rows_a)
        pltpu.sync_copy(rows_a, out_hbm.at[pl.ds(cid * _NACC + r0, 2 * _C)])


def _sc_aggregate(a0, a1, rf, eg):
    mesh = plsc.VectorSubcoreMesh(core_axis_name="c", subcore_axis_name="s")
    k = pl.kernel(
        _sc_body,
        out_type=jax.ShapeDtypeStruct((2 * _NACC, 128), jnp.float32),
        mesh=mesh,
        scratch_types=[
            pltpu.VMEM((2 * _C,), jnp.int32),
            pltpu.VMEM((2 * _C, 128), jnp.float32),
            pltpu.VMEM((_C, 128), jnp.float32),
            pltpu.VMEM((2 * _C,), jnp.int32),
            pltpu.VMEM((2 * _C, 128), jnp.float32),
            pltpu.VMEM((_C, 128), jnp.float32),
            pltpu.VMEM_SHARED((_NACC, 128), jnp.float32),
            pltpu.SemaphoreType.DMA,
            pltpu.SemaphoreType.DMA,
        ],
    )
    return k(a0, a1, rf, eg)


_BU = 400


def _update_body(p_ref, w1_ref, b1_ref, w2_ref, b2_ref, out_ref):
    agg = p_ref[0] + p_ref[1]
    h = jnp.dot(agg, w1_ref[...], preferred_element_type=jnp.float32) + b1_ref[...]
    h = jax.nn.softplus(h) - LOG2
    out_ref[...] = (
        jnp.dot(h, w2_ref[...], preferred_element_type=jnp.float32) + b2_ref[...]
    )


def _node_update(parts, W_d1, b_d1, W_d2, b_d2, N):
    grid = N // _BU
    return pl.pallas_call(
        _update_body,
        grid=(grid,),
        in_specs=[
            pl.BlockSpec((2, _BU, 128), lambda i: (0, i, 0)),
            pl.BlockSpec((128, 128), lambda i: (0, 0)),
            pl.BlockSpec((1, 128), lambda i: (0, 0)),
            pl.BlockSpec((128, 128), lambda i: (0, 0)),
            pl.BlockSpec((1, 128), lambda i: (0, 0)),
        ],
        out_specs=pl.BlockSpec((_BU, 128), lambda i: (i, 0)),
        out_shape=jax.ShapeDtypeStruct((N, 128), jnp.float32),
    )(parts, W_d1, b_d1.reshape(1, 128), W_d2, b_d2.reshape(1, 128))


@jax.jit
def kernel(r, e, a, W_df1, b_df1, W_df2, b_df2, W_af, W_d1, b_d1, W_d2, b_d2):
    N = r.shape[0]
    eg = _edge_filters(e, W_df1, b_df1, W_df2, b_df2)
    rf = _atom_filter(r, W_af)
    parts_flat = _sc_aggregate(a[:, 0], a[:, 1], rf, eg)
    parts = parts_flat.reshape(2, _NACC, 128)
    return _node_update(parts, W_d1, b_d1, W_d2, b_d2, N)


# speedup vs baseline: 1.8047x; 1.0002x over previous
import functools

import jax
import jax.numpy as jnp
import numpy as np
from jax import lax
from jax.experimental import pallas as pl
from jax.experimental.pallas import tpu as pltpu
from jax.experimental.pallas import tpu_sc as plsc

N_G = 50
CUT = 5.0
LOG2 = 0.6931471805599453


_BE = 3200


def _edge_filter_body(e_ref, w1_ref, b1_ref, w2_ref, b2_ref, out_ref):
    width = CUT / (N_G - 1)
    coeff = -0.5 / (width * width)
    offs = lax.broadcasted_iota(jnp.int32, (1, N_G), 1).astype(jnp.float32) * width
    e = e_ref[...]
    d = e - offs
    eg = jnp.exp(coeff * d * d)
    h = jnp.dot(eg, w1_ref[...], preferred_element_type=jnp.float32) + b1_ref[...]
    h = jax.nn.softplus(h) - LOG2
    out_ref[...] = (
        jnp.dot(h, w2_ref[...], preferred_element_type=jnp.float32) + b2_ref[...]
    )


def _edge_filters(e, W_df1, b_df1, W_df2, b_df2):
    E = e.shape[0]
    grid = E // _BE
    return pl.pallas_call(
        _edge_filter_body,
        grid=(grid,),
        in_specs=[
            pl.BlockSpec((_BE, 1), lambda i: (i, 0)),
            pl.BlockSpec((N_G, N_G), lambda i: (0, 0)),
            pl.BlockSpec((1, N_G), lambda i: (0, 0)),
            pl.BlockSpec((N_G, 128), lambda i: (0, 0)),
            pl.BlockSpec((1, 128), lambda i: (0, 0)),
        ],
        out_specs=pl.BlockSpec((_BE, 128), lambda i: (i, 0)),
        out_shape=jax.ShapeDtypeStruct((E, 128), jnp.float32),
    )(e, W_df1, b_df1.reshape(1, N_G), W_df2, b_df2.reshape(1, 128))


_BN = 2000


def _atom_filter_body(r_ref, w_ref, out_ref):
    out_ref[...] = jnp.dot(r_ref[...], w_ref[...], preferred_element_type=jnp.float32)


def _atom_filter(r, W_af):
    N = r.shape[0]
    grid = N // _BN
    return pl.pallas_call(
        _atom_filter_body,
        grid=(grid,),
        in_specs=[
            pl.BlockSpec((_BN, 128), lambda i: (i, 0)),
            pl.BlockSpec((128, 128), lambda i: (0, 0)),
        ],
        out_specs=pl.BlockSpec((_BN, 128), lambda i: (i, 0)),
        out_shape=jax.ShapeDtypeStruct((N, 128), jnp.float32),
    )(r, W_af)


_C = 40
_NACC = 10000
_EPT = 10000
_CPT = _EPT // _C
_NBLK = _NACC // _C


def _sc_body(a0_hbm, a1_hbm, rf_hbm, eg_hbm, out_hbm,
             sidx_a, rows_a, eg_a,
             sidx_b, rows_b, eg_b,
             acc_sh, sem_a, sem_b, ssem_a, ssem_b):
    cid = lax.axis_index("c")
    sid = lax.axis_index("s")
    wid = sid * 2 + cid
    ebase = wid * _EPT

    bufs_a = (sidx_a, rows_a, eg_a, sem_a, ssem_a)
    bufs_b = (sidx_b, rows_b, eg_b, sem_b, ssem_b)

    def drain_scatter(bufs):
        sidx, rows, egb, sem, ssem = bufs
        pltpu.make_async_copy(rows, acc_sh.at[sidx], ssem).wait()

    def fire(c, bufs, drain):
        sidx, rows, egb, sem, ssem = bufs
        base = ebase + c * _C
        if drain:
            drain_scatter(bufs)
        pltpu.sync_copy(a1_hbm.at[pl.ds(base, _C)], sidx.at[pl.ds(0, _C)])
        pltpu.sync_copy(a0_hbm.at[pl.ds(base, _C)], sidx.at[pl.ds(_C, _C)])
        pltpu.async_copy(rf_hbm.at[sidx.at[pl.ds(_C, _C)]], rows.at[pl.ds(0, _C)], sem)
        pltpu.async_copy(rf_hbm.at[sidx.at[pl.ds(0, _C)]], rows.at[pl.ds(_C, _C)], sem)
        pltpu.async_copy(eg_hbm.at[pl.ds(base, _C)], egb, sem)

    def process(c, bufs):
        sidx, rows, egb, sem, ssem = bufs
        pltpu.make_async_copy(eg_hbm.at[pl.ds(0, _C)], rows.at[pl.ds(0, _C)], sem).wait()
        pltpu.make_async_copy(eg_hbm.at[pl.ds(0, _C)], rows.at[pl.ds(_C, _C)], sem).wait()
        pltpu.make_async_copy(eg_hbm.at[pl.ds(0, _C)], egb, sem).wait()

        @pl.loop(0, _C)
        def _(i):
            for j in range(8):
                s = pl.ds(j * 16, 16)
                eij = egb[i, s]
                rows[i, s] = rows[i, s] * eij
                rows[_C + i, s] = rows[_C + i, s] * eij

        pltpu.async_copy(rows, acc_sh.at[sidx], ssem, add=True)

    zeros16 = jnp.zeros((16,), jnp.float32)

    @pl.loop(0, 2 * _C)
    def _(i):
        for j in range(8):
            rows_a[i, pl.ds(j * 16, 16)] = zeros16

    nz = (_NACC // (2 * _C) - sid + 15) // 16

    @pl.loop(0, nz)
    def _(k):
        pltpu.sync_copy(rows_a, acc_sh.at[pl.ds((sid + k * 16) * 2 * _C, 2 * _C)])

    plsc.subcore_barrier()

    fire(0, bufs_a, False)
    fire(1, bufs_b, False)
    process(0, bufs_a)
    fire(2, bufs_a, True)
    process(1, bufs_b)
    fire(3, bufs_b, True)

    @pl.loop(1, _CPT // 2 - 1)
    def _(kk):
        process(2 * kk, bufs_a)
        fire(2 * kk + 2, bufs_a, True)
        process(2 * kk + 1, bufs_b)
        fire(2 * kk + 3, bufs_b, True)

    process(_CPT - 2, bufs_a)
    process(_CPT - 1, bufs_b)
    drain_scatter(bufs_a)
    drain_scatter(bufs_b)

    plsc.subcore_barrier()

    @pl.loop(0, nz)
    def _(k):
        r0 = (sid + k * 16) * 2 * _C
        pltpu.sync_copy(acc_sh.at[pl.ds(r0, 2 * _C)], rows_a)
        pltpu.sync_copy(rows_a, out_hbm.at[pl.ds(cid * _NACC + r0, 2 * _C)])


def _sc_aggregate(a0, a1, rf, eg):
    mesh = plsc.VectorSubcoreMesh(core_axis_name="c", subcore_axis_name="s")
    k = pl.kernel(
        _sc_body,
        out_type=jax.ShapeDtypeStruct((2 * _NACC, 128), jnp.float32),
        mesh=mesh,
        scratch_types=[
            pltpu.VMEM((2 * _C,), jnp.int32),
            pltpu.VMEM((2 * _C, 128), jnp.float32),
            pltpu.VMEM((_C, 128), jnp.float32),
            pltpu.VMEM((2 * _C,), jnp.int32),
            pltpu.VMEM((2 * _C, 128), jnp.float32),
            pltpu.VMEM((_C, 128), jnp.float32),
            pltpu.VMEM_SHARED((_NACC, 128), jnp.float32),
            pltpu.SemaphoreType.DMA,
            pltpu.SemaphoreType.DMA,
            pltpu.SemaphoreType.DMA,
            pltpu.SemaphoreType.DMA,
        ],
    )
    return k(a0, a1, rf, eg)


_BU = 400


def _update_body(p_ref, w1_ref, b1_ref, w2_ref, b2_ref, out_ref):
    agg = p_ref[0] + p_ref[1]
    h = jnp.dot(agg, w1_ref[...], preferred_element_type=jnp.float32) + b1_ref[...]
    h = jax.nn.softplus(h) - LOG2
    out_ref[...] = (
        jnp.dot(h, w2_ref[...], preferred_element_type=jnp.float32) + b2_ref[...]
    )


def _node_update(parts, W_d1, b_d1, W_d2, b_d2, N):
    grid = N // _BU
    return pl.pallas_call(
        _update_body,
        grid=(grid,),
        in_specs=[
            pl.BlockSpec((2, _BU, 128), lambda i: (0, i, 0)),
            pl.BlockSpec((128, 128), lambda i: (0, 0)),
            pl.BlockSpec((1, 128), lambda i: (0, 0)),
            pl.BlockSpec((128, 128), lambda i: (0, 0)),
            pl.BlockSpec((1, 128), lambda i: (0, 0)),
        ],
        out_specs=pl.BlockSpec((_BU, 128), lambda i: (i, 0)),
        out_shape=jax.ShapeDtypeStruct((N, 128), jnp.float32),
    )(parts, W_d1, b_d1.reshape(1, 128), W_d2, b_d2.reshape(1, 128))


@jax.jit
def kernel(r, e, a, W_df1, b_df1, W_df2, b_df2, W_af, W_d1, b_d1, W_d2, b_d2):
    N = r.shape[0]
    eg = _edge_filters(e, W_df1, b_df1, W_df2, b_df2)
    rf = _atom_filter(r, W_af)
    parts_flat = _sc_aggregate(a[:, 0], a[:, 1], rf, eg)
    parts = parts_flat.reshape(2, _NACC, 128)
    return _node_update(parts, W_d1, b_d1, W_d2, b_d2, N)


# speedup vs baseline: 1.9986x; 1.1074x over previous
import functools

import jax
import jax.numpy as jnp
import numpy as np
from jax import lax
from jax.experimental import pallas as pl
from jax.experimental.pallas import tpu as pltpu
from jax.experimental.pallas import tpu_sc as plsc

N_G = 50
CUT = 5.0
LOG2 = 0.6931471805599453


_BE = 3200


def _edge_filter_body(e_ref, w1_ref, b1_ref, w2_ref, b2_ref, out_ref):
    width = CUT / (N_G - 1)
    coeff = -0.5 / (width * width)
    offs = lax.broadcasted_iota(jnp.int32, (1, N_G), 1).astype(jnp.float32) * width
    e = e_ref[...]
    d = e - offs
    eg = jnp.exp(coeff * d * d)
    h = jnp.dot(eg, w1_ref[...], preferred_element_type=jnp.float32) + b1_ref[...]
    h = jax.nn.softplus(h) - LOG2
    out_ref[...] = (
        jnp.dot(h, w2_ref[...], preferred_element_type=jnp.float32) + b2_ref[...]
    )


def _edge_filters(e, W_df1, b_df1, W_df2, b_df2):
    E = e.shape[0]
    grid = E // _BE
    return pl.pallas_call(
        _edge_filter_body,
        grid=(grid,),
        in_specs=[
            pl.BlockSpec((_BE, 1), lambda i: (i, 0)),
            pl.BlockSpec((N_G, N_G), lambda i: (0, 0)),
            pl.BlockSpec((1, N_G), lambda i: (0, 0)),
            pl.BlockSpec((N_G, 128), lambda i: (0, 0)),
            pl.BlockSpec((1, 128), lambda i: (0, 0)),
        ],
        out_specs=pl.BlockSpec((_BE, 128), lambda i: (i, 0)),
        out_shape=jax.ShapeDtypeStruct((E, 128), jnp.float32),
    )(e, W_df1, b_df1.reshape(1, N_G), W_df2, b_df2.reshape(1, 128))


_BN = 2000


def _atom_filter_body(r_ref, w_ref, out_ref):
    out_ref[...] = jnp.dot(r_ref[...], w_ref[...], preferred_element_type=jnp.float32)


def _atom_filter(r, W_af):
    N = r.shape[0]
    grid = N // _BN
    return pl.pallas_call(
        _atom_filter_body,
        grid=(grid,),
        in_specs=[
            pl.BlockSpec((_BN, 128), lambda i: (i, 0)),
            pl.BlockSpec((128, 128), lambda i: (0, 0)),
        ],
        out_specs=pl.BlockSpec((_BN, 128), lambda i: (i, 0)),
        out_shape=jax.ShapeDtypeStruct((N, 128), jnp.float32),
    )(r, W_af)


_C = 40
_NACC = 10000
_ESPLIT = 166400


def _sc_body(ept, a0_hbm, a1_hbm, rf_hbm, eg_hbm, out_hbm,
             sidx_a, rows_a, eg_a,
             sidx_b, rows_b, eg_b,
             acc_sh, sem_a, sem_b, ssem_a, ssem_b):
    cid = lax.axis_index("c")
    sid = lax.axis_index("s")
    wid = sid * 2 + cid
    ebase = wid * ept
    cpt = ept // _C

    bufs_a = (sidx_a, rows_a, eg_a, sem_a, ssem_a)
    bufs_b = (sidx_b, rows_b, eg_b, sem_b, ssem_b)

    def drain_scatter(bufs):
        sidx, rows, egb, sem, ssem = bufs
        pltpu.make_async_copy(rows, acc_sh.at[sidx], ssem).wait()

    def fire(c, bufs, drain):
        sidx, rows, egb, sem, ssem = bufs
        base = ebase + c * _C
        if drain:
            drain_scatter(bufs)
        pltpu.sync_copy(a1_hbm.at[pl.ds(base, _C)], sidx.at[pl.ds(0, _C)])
        pltpu.sync_copy(a0_hbm.at[pl.ds(base, _C)], sidx.at[pl.ds(_C, _C)])
        pltpu.async_copy(rf_hbm.at[sidx.at[pl.ds(_C, _C)]], rows.at[pl.ds(0, _C)], sem)
        pltpu.async_copy(rf_hbm.at[sidx.at[pl.ds(0, _C)]], rows.at[pl.ds(_C, _C)], sem)
        pltpu.async_copy(eg_hbm.at[pl.ds(base, _C)], egb, sem)

    def process(c, bufs):
        sidx, rows, egb, sem, ssem = bufs
        pltpu.make_async_copy(eg_hbm.at[pl.ds(0, _C)], rows.at[pl.ds(0, _C)], sem).wait()
        pltpu.make_async_copy(eg_hbm.at[pl.ds(0, _C)], rows.at[pl.ds(_C, _C)], sem).wait()
        pltpu.make_async_copy(eg_hbm.at[pl.ds(0, _C)], egb, sem).wait()

        @pl.loop(0, _C)
        def _(i):
            for j in range(8):
                s = pl.ds(j * 16, 16)
                eij = egb[i, s]
                rows[i, s] = rows[i, s] * eij
                rows[_C + i, s] = rows[_C + i, s] * eij

        pltpu.async_copy(rows, acc_sh.at[sidx], ssem, add=True)

    zeros16 = jnp.zeros((16,), jnp.float32)

    @pl.loop(0, 2 * _C)
    def _(i):
        for j in range(8):
            rows_a[i, pl.ds(j * 16, 16)] = zeros16

    nz = (_NACC // (2 * _C) - sid + 15) // 16

    @pl.loop(0, nz)
    def _(k):
        pltpu.sync_copy(rows_a, acc_sh.at[pl.ds((sid + k * 16) * 2 * _C, 2 * _C)])

    plsc.subcore_barrier()

    fire(0, bufs_a, False)
    fire(1, bufs_b, False)
    process(0, bufs_a)
    fire(2, bufs_a, True)
    process(1, bufs_b)
    fire(3, bufs_b, True)

    @pl.loop(1, cpt // 2 - 1)
    def _(kk):
        process(2 * kk, bufs_a)
        fire(2 * kk + 2, bufs_a, True)
        process(2 * kk + 1, bufs_b)
        fire(2 * kk + 3, bufs_b, True)

    process(cpt - 2, bufs_a)
    process(cpt - 1, bufs_b)
    drain_scatter(bufs_a)
    drain_scatter(bufs_b)

    plsc.subcore_barrier()

    @pl.loop(0, nz)
    def _(k):
        r0 = (sid + k * 16) * 2 * _C
        pltpu.sync_copy(acc_sh.at[pl.ds(r0, 2 * _C)], rows_a)
        pltpu.sync_copy(rows_a, out_hbm.at[pl.ds(cid * _NACC + r0, 2 * _C)])


def _sc_aggregate(a0, a1, rf, eg):
    ept = a0.shape[0] // 32
    mesh = plsc.VectorSubcoreMesh(core_axis_name="c", subcore_axis_name="s")
    k = pl.kernel(
        functools.partial(_sc_body, ept),
        out_type=jax.ShapeDtypeStruct((2 * _NACC, 128), jnp.float32),
        mesh=mesh,
        scratch_types=[
            pltpu.VMEM((2 * _C,), jnp.int32),
            pltpu.VMEM((2 * _C, 128), jnp.float32),
            pltpu.VMEM((_C, 128), jnp.float32),
            pltpu.VMEM((2 * _C,), jnp.int32),
            pltpu.VMEM((2 * _C, 128), jnp.float32),
            pltpu.VMEM((_C, 128), jnp.float32),
            pltpu.VMEM_SHARED((_NACC, 128), jnp.float32),
            pltpu.SemaphoreType.DMA,
            pltpu.SemaphoreType.DMA,
            pltpu.SemaphoreType.DMA,
            pltpu.SemaphoreType.DMA,
        ],
    )
    return k(a0, a1, rf, eg)


_BU = 400


def _update_body(p_ref, w1_ref, b1_ref, w2_ref, b2_ref, out_ref):
    agg = (p_ref[0] + p_ref[1]) + (p_ref[2] + p_ref[3])
    h = jnp.dot(agg, w1_ref[...], preferred_element_type=jnp.float32) + b1_ref[...]
    h = jax.nn.softplus(h) - LOG2
    out_ref[...] = (
        jnp.dot(h, w2_ref[...], preferred_element_type=jnp.float32) + b2_ref[...]
    )


def _node_update(parts, W_d1, b_d1, W_d2, b_d2, N):
    grid = N // _BU
    return pl.pallas_call(
        _update_body,
        grid=(grid,),
        in_specs=[
            pl.BlockSpec((4, _BU, 128), lambda i: (0, i, 0)),
            pl.BlockSpec((128, 128), lambda i: (0, 0)),
            pl.BlockSpec((1, 128), lambda i: (0, 0)),
            pl.BlockSpec((128, 128), lambda i: (0, 0)),
            pl.BlockSpec((1, 128), lambda i: (0, 0)),
        ],
        out_specs=pl.BlockSpec((_BU, 128), lambda i: (i, 0)),
        out_shape=jax.ShapeDtypeStruct((N, 128), jnp.float32),
    )(parts, W_d1, b_d1.reshape(1, 128), W_d2, b_d2.reshape(1, 128))


@jax.jit
def kernel(r, e, a, W_df1, b_df1, W_df2, b_df2, W_af, W_d1, b_d1, W_d2, b_d2):
    N = r.shape[0]
    rf = _atom_filter(r, W_af)
    eg0 = _edge_filters(e[:_ESPLIT], W_df1, b_df1, W_df2, b_df2)
    eg1 = _edge_filters(e[_ESPLIT:], W_df1, b_df1, W_df2, b_df2)
    p0 = _sc_aggregate(a[:_ESPLIT, 0], a[:_ESPLIT, 1], rf, eg0)
    p1 = _sc_aggregate(a[_ESPLIT:, 0], a[_ESPLIT:, 1], rf, eg1)
    parts = jnp.concatenate(
        [p0.reshape(2, _NACC, 128), p1.reshape(2, _NACC, 128)], axis=0
    )
    return _node_update(parts, W_d1, b_d1, W_d2, b_d2, N)


# speedup vs baseline: 2.4711x; 1.2364x over previous
import functools

import jax
import jax.numpy as jnp
import numpy as np
from jax import lax
from jax.experimental import pallas as pl
from jax.experimental.pallas import tpu as pltpu
from jax.experimental.pallas import tpu_sc as plsc

N_G = 50
CUT = 5.0
LOG2 = 0.6931471805599453


_BE = 3200


def _edge_filter_body(e_ref, w1_ref, b1_ref, w2_ref, b2_ref, out_ref):
    width = CUT / (N_G - 1)
    coeff = -0.5 / (width * width)
    offs = lax.broadcasted_iota(jnp.int32, (1, N_G), 1).astype(jnp.float32) * width
    e = e_ref[...]
    d = e - offs
    eg = jnp.exp(coeff * d * d)
    h = jnp.dot(eg, w1_ref[...], preferred_element_type=jnp.float32) + b1_ref[...]
    h = jax.nn.softplus(h) - LOG2
    out_ref[...] = (
        jnp.dot(h, w2_ref[...], preferred_element_type=jnp.float32) + b2_ref[...]
    )


def _edge_filters(e, W_df1, b_df1, W_df2, b_df2):
    E = e.shape[0]
    grid = E // _BE
    return pl.pallas_call(
        _edge_filter_body,
        grid=(grid,),
        in_specs=[
            pl.BlockSpec((_BE, 1), lambda i: (i, 0)),
            pl.BlockSpec((N_G, N_G), lambda i: (0, 0)),
            pl.BlockSpec((1, N_G), lambda i: (0, 0)),
            pl.BlockSpec((N_G, 128), lambda i: (0, 0)),
            pl.BlockSpec((1, 128), lambda i: (0, 0)),
        ],
        out_specs=pl.BlockSpec((_BE, 128), lambda i: (i, 0)),
        out_shape=jax.ShapeDtypeStruct((E, 128), jnp.float32),
    )(e, W_df1, b_df1.reshape(1, N_G), W_df2, b_df2.reshape(1, 128))


_BN = 2000


def _atom_filter_body(r_ref, w_ref, out_ref):
    out_ref[...] = jnp.dot(r_ref[...], w_ref[...], preferred_element_type=jnp.float32)


def _atom_filter(r, W_af):
    N = r.shape[0]
    grid = N // _BN
    return pl.pallas_call(
        _atom_filter_body,
        grid=(grid,),
        in_specs=[
            pl.BlockSpec((_BN, 128), lambda i: (i, 0)),
            pl.BlockSpec((128, 128), lambda i: (0, 0)),
        ],
        out_specs=pl.BlockSpec((_BN, 128), lambda i: (i, 0)),
        out_shape=jax.ShapeDtypeStruct((N, 128), jnp.float32),
    )(r, W_af)


_C = 40
_NACC = 10000
_ESPLIT = 166400


def _sc_body(ept, ap_hbm, rf_hbm, eg_hbm, out_hbm,
             ap_f, sidx_a, rows_a, eg_a,
             sidx_b, rows_b, eg_b,
             acc_sh, sem_a, sem_b, ssem_a, ssem_b):
    cid = lax.axis_index("c")
    sid = lax.axis_index("s")
    wid = sid * 2 + cid
    ebase = wid * ept
    cpt = ept // _C

    bufs_a = (sidx_a, rows_a, eg_a, sem_a, ssem_a)
    bufs_b = (sidx_b, rows_b, eg_b, sem_b, ssem_b)

    def drain_scatter(bufs):
        sidx, rows, egb, sem, ssem = bufs
        pltpu.make_async_copy(rows, acc_sh.at[sidx], ssem).wait()

    def fire(c, bufs, drain):
        sidx, rows, egb, sem, ssem = bufs
        base = ebase + c * _C
        if drain:
            drain_scatter(bufs)
        for off in (0, 16, 24):
            w = ap_f[pl.ds(c * _C + off, 16)]
            sidx[pl.ds(off, 16)] = (w >> 16).astype(jnp.int32)
            sidx[pl.ds(_C + off, 16)] = (w & 0xFFFF).astype(jnp.int32)
        pltpu.async_copy(rf_hbm.at[sidx.at[pl.ds(_C, _C)]], rows.at[pl.ds(0, _C)], sem)
        pltpu.async_copy(rf_hbm.at[sidx.at[pl.ds(0, _C)]], rows.at[pl.ds(_C, _C)], sem)
        pltpu.async_copy(eg_hbm.at[pl.ds(base, _C)], egb, sem)

    def process(c, bufs):
        sidx, rows, egb, sem, ssem = bufs
        pltpu.make_async_copy(eg_hbm.at[pl.ds(0, _C)], rows.at[pl.ds(0, _C)], sem).wait()
        pltpu.make_async_copy(eg_hbm.at[pl.ds(0, _C)], rows.at[pl.ds(_C, _C)], sem).wait()
        pltpu.make_async_copy(eg_hbm.at[pl.ds(0, _C)], egb, sem).wait()

        @pl.loop(0, _C)
        def _(i):
            for j in range(8):
                s = pl.ds(j * 16, 16)
                eij = egb[i, s]
                rows[i, s] = rows[i, s] * eij
                rows[_C + i, s] = rows[_C + i, s] * eij

        pltpu.async_copy(rows, acc_sh.at[sidx], ssem, add=True)

    pltpu.sync_copy(ap_hbm.at[pl.ds(wid * ept, ept)], ap_f)

    zeros16 = jnp.zeros((16,), jnp.float32)

    @pl.loop(0, 2 * _C)
    def _(i):
        for j in range(8):
            rows_a[i, pl.ds(j * 16, 16)] = zeros16

    nz = (_NACC // (2 * _C) - sid + 15) // 16

    @pl.loop(0, nz)
    def _(k):
        pltpu.sync_copy(rows_a, acc_sh.at[pl.ds((sid + k * 16) * 2 * _C, 2 * _C)])

    plsc.subcore_barrier()

    fire(0, bufs_a, False)
    fire(1, bufs_b, False)
    process(0, bufs_a)
    fire(2, bufs_a, True)
    process(1, bufs_b)
    fire(3, bufs_b, True)

    @pl.loop(1, cpt // 2 - 1)
    def _(kk):
        process(2 * kk, bufs_a)
        fire(2 * kk + 2, bufs_a, True)
        process(2 * kk + 1, bufs_b)
        fire(2 * kk + 3, bufs_b, True)

    process(cpt - 2, bufs_a)
    process(cpt - 1, bufs_b)
    drain_scatter(bufs_a)
    drain_scatter(bufs_b)

    plsc.subcore_barrier()

    @pl.loop(0, nz)
    def _(k):
        r0 = (sid + k * 16) * 2 * _C
        pltpu.sync_copy(acc_sh.at[pl.ds(r0, 2 * _C)], rows_a)
        pltpu.sync_copy(rows_a, out_hbm.at[pl.ds(cid * _NACC + r0, 2 * _C)])


def _sc_aggregate(ap, rf, eg):
    ept = ap.shape[0] // 32
    mesh = plsc.VectorSubcoreMesh(core_axis_name="c", subcore_axis_name="s")
    k = pl.kernel(
        functools.partial(_sc_body, ept),
        out_type=jax.ShapeDtypeStruct((2 * _NACC, 128), jnp.float32),
        mesh=mesh,
        scratch_types=[
            pltpu.VMEM((ept,), jnp.uint32),
            pltpu.VMEM((2 * _C,), jnp.int32),
            pltpu.VMEM((2 * _C, 128), jnp.float32),
            pltpu.VMEM((_C, 128), jnp.float32),
            pltpu.VMEM((2 * _C,), jnp.int32),
            pltpu.VMEM((2 * _C, 128), jnp.float32),
            pltpu.VMEM((_C, 128), jnp.float32),
            pltpu.VMEM_SHARED((_NACC, 128), jnp.float32),
            pltpu.SemaphoreType.DMA,
            pltpu.SemaphoreType.DMA,
            pltpu.SemaphoreType.DMA,
            pltpu.SemaphoreType.DMA,
        ],
    )
    return k(ap, rf, eg)


_BU = 400


def _update_body(p_ref, w1_ref, b1_ref, w2_ref, b2_ref, out_ref):
    agg = (p_ref[0] + p_ref[1]) + (p_ref[2] + p_ref[3])
    h = jnp.dot(agg, w1_ref[...], preferred_element_type=jnp.float32) + b1_ref[...]
    h = jax.nn.softplus(h) - LOG2
    out_ref[...] = (
        jnp.dot(h, w2_ref[...], preferred_element_type=jnp.float32) + b2_ref[...]
    )


def _node_update(parts, W_d1, b_d1, W_d2, b_d2, N):
    grid = N // _BU
    return pl.pallas_call(
        _update_body,
        grid=(grid,),
        in_specs=[
            pl.BlockSpec((4, _BU, 128), lambda i: (0, i, 0)),
            pl.BlockSpec((128, 128), lambda i: (0, 0)),
            pl.BlockSpec((1, 128), lambda i: (0, 0)),
            pl.BlockSpec((128, 128), lambda i: (0, 0)),
            pl.BlockSpec((1, 128), lambda i: (0, 0)),
        ],
        out_specs=pl.BlockSpec((_BU, 128), lambda i: (i, 0)),
        out_shape=jax.ShapeDtypeStruct((N, 128), jnp.float32),
    )(parts, W_d1, b_d1.reshape(1, 128), W_d2, b_d2.reshape(1, 128))


@jax.jit
def kernel(r, e, a, W_df1, b_df1, W_df2, b_df2, W_af, W_d1, b_d1, W_d2, b_d2):
    N = r.shape[0]
    rf = _atom_filter(r, W_af)
    eg0 = _edge_filters(e[:_ESPLIT], W_df1, b_df1, W_df2, b_df2)
    eg1 = _edge_filters(e[_ESPLIT:], W_df1, b_df1, W_df2, b_df2)
    ap = a[:, 0].astype(jnp.uint32) | (a[:, 1].astype(jnp.uint32) << 16)
    p0 = _sc_aggregate(ap[:_ESPLIT], rf, eg0)
    p1 = _sc_aggregate(ap[_ESPLIT:], rf, eg1)
    parts = jnp.concatenate(
        [p0.reshape(2, _NACC, 128), p1.reshape(2, _NACC, 128)], axis=0
    )
    return _node_update(parts, W_d1, b_d1, W_d2, b_d2, N)
